# baseline (speedup 1.0000x reference)
import jax
import jax.numpy as jnp
from jax.experimental import pallas as pl
from jax.experimental.pallas import tpu as pltpu

B = 1024
F = 81920
CHUNK = 2048
NBUF = 3
NCH = F // CHUNK


def _nnue_kernel(white_hbm, black_hbm, turn_ref, score_ref,
                 w0_ref, b0_ref, w1_ref, b1_ref, w2_ref, b2_ref,
                 loss_ref, wbuf, bbuf, accw_ref, accb_ref, wsem, bsem):
    def wcopy(k):
        s = k % NBUF
        return pltpu.make_async_copy(
            white_hbm.at[:, pl.ds(k * CHUNK, CHUNK)], wbuf.at[s], wsem.at[s])

    def bcopy(k):
        s = k % NBUF
        return pltpu.make_async_copy(
            black_hbm.at[:, pl.ds(k * CHUNK, CHUNK)], bbuf.at[s], bsem.at[s])

    for k in range(NBUF - 1):
        wcopy(k).start()
        bcopy(k).start()

    accw_ref[...] = jnp.zeros_like(accw_ref)
    accb_ref[...] = jnp.zeros_like(accb_ref)

    dn = (((1,), (1,)), ((), ()))
    for k in range(NCH):
        s = k % NBUF
        wcopy(k).wait()
        bcopy(k).wait()
        if k + NBUF - 1 < NCH:
            wcopy(k + NBUF - 1).start()
            bcopy(k + NBUF - 1).start()
        w0b = w0_ref[:, k * CHUNK:(k + 1) * CHUNK].astype(jnp.bfloat16)
        accw_ref[...] += jax.lax.dot_general(
            wbuf[s].astype(jnp.bfloat16), w0b, dn,
            preferred_element_type=jnp.float32)
        accb_ref[...] += jax.lax.dot_general(
            bbuf[s].astype(jnp.bfloat16), w0b, dn,
            preferred_element_type=jnp.float32)

    b0 = b0_ref[...]
    w = accw_ref[...] + b0
    b = accb_ref[...] + b0
    turn = turn_ref[...]
    wb = jnp.concatenate([w, b], axis=1)
    bw = jnp.concatenate([b, w], axis=1)
    accum = turn * wb + (1.0 - turn) * bw
    l1_x = jnp.clip(accum, 0.0, 1.0)
    l2 = jax.lax.dot_general(l1_x, w1_ref[...], dn,
                             preferred_element_type=jnp.float32) + b1_ref[...]
    l2_x = jnp.clip(l2, 0.0, 1.0)
    model = jnp.sum(l2_x * w2_ref[...], axis=1, keepdims=True) + b2_ref[...]
    wdl_model = jax.nn.sigmoid(model / 400.0)
    wdl_target = jax.nn.sigmoid(score_ref[...] / 400.0)
    loss_ref[...] = (wdl_model - wdl_target) ** 2


@jax.jit
def _nnue(white_features, black_features, turn, score,
          W0, b0, W1, b1, W2, b2):
    hbm = pl.BlockSpec(memory_space=pltpu.MemorySpace.HBM)
    return pl.pallas_call(
        _nnue_kernel,
        in_specs=[hbm, hbm] + [pl.BlockSpec(memory_space=pltpu.MemorySpace.VMEM)] * 8,
        out_specs=pl.BlockSpec(memory_space=pltpu.MemorySpace.VMEM),
        out_shape=jax.ShapeDtypeStruct((B, 1), jnp.float32),
        scratch_shapes=[
            pltpu.VMEM((NBUF, B, CHUNK), jnp.float32),
            pltpu.VMEM((NBUF, B, CHUNK), jnp.float32),
            pltpu.VMEM((B, 4), jnp.float32),
            pltpu.VMEM((B, 4), jnp.float32),
            pltpu.SemaphoreType.DMA((NBUF,)),
            pltpu.SemaphoreType.DMA((NBUF,)),
        ],
    )(white_features, black_features, turn, score,
      W0, b0, W1, b1, W2, b2)


def kernel(white_features, black_features, turn, score, result,
           W0, b0, W1, b1, W2, b2):
    del result
    return _nnue(white_features, black_features, turn, score,
                 W0, b0.reshape(1, 4), W1, b1.reshape(1, 8),
                 W2.reshape(1, 8), b2.reshape(1, 1))
